# Initial kernel scaffold; baseline (speedup 1.0000x reference)
import jax
import jax.numpy as jnp
from jax.experimental import pallas as pl


def kernel(x):
    raise NotImplementedError("write your pallas kernel here")


# speedup vs baseline: 1.0868x; 1.0868x over previous
import functools

import jax
import jax.numpy as jnp
from jax.experimental import pallas as pl
from jax.experimental.pallas import tpu as pltpu

_BINS = 10
_K = 10


def _stats_body(x_ref, s_ref):
    xb = x_ref[0]
    n = xb.shape[0] * xb.shape[1]
    mean = jnp.sum(xb) / n
    d = xb - mean
    var = jnp.sum(d * d) / (n - 1)
    q = jnp.round(jnp.tanh(xb) * 10.0)
    lane = jax.lax.broadcasted_iota(jnp.int32, (1, 16), 1)
    vec = jnp.where(lane == 0, var, 0.0)
    for v in range(_BINS + 1):
        cnt = jnp.sum(jnp.where(q == float(v), 1.0, 0.0))
        vec = jnp.where(lane == 1 + v, cnt, vec)
    s_ref[0] = vec


def _select_body(st_ref, m_ref):
    st = st_ref[...]
    b, c, _ = st.shape
    var = st[:, :, 0]
    hs = st[:, :, 1:_BINS + 2]
    total = jnp.sum(hs, axis=2, keepdims=True)
    p = hs / total
    min_real = jnp.finfo(jnp.float32).min
    logit = jnp.maximum(jnp.log(p), min_real)
    ent = -jnp.sum(logit * p, axis=2)
    score = ent + 2.0 / (var + 1e-7)
    si = score[:, :, None]
    sj = score[:, None, :]
    jidx = jax.lax.broadcasted_iota(jnp.int32, (b, c, c), 2)
    iidx = jax.lax.broadcasted_iota(jnp.int32, (b, c, c), 1)
    ahead = (sj > si) | ((sj == si) & (jidx < iidx))
    rank = jnp.sum(ahead.astype(jnp.int32), axis=2)
    m_ref[...] = (rank < _K).astype(jnp.int32)


def _conv_body(x_ref, m_ref, o_ref):
    i = pl.program_id(0)
    sel = m_ref[i]
    xb = x_ref[0]
    h, w = xb.shape

    @pl.when(sel > 0)
    def _lap():
        zc = jnp.zeros((h, 1), jnp.float32)
        hs = xb + jnp.concatenate([xb[:, 1:], zc], axis=1) \
                + jnp.concatenate([zc, xb[:, :w - 1]], axis=1)
        zr = jnp.zeros((1, w), jnp.float32)
        box = hs + jnp.concatenate([hs[1:, :], zr], axis=0) \
                 + jnp.concatenate([zr, hs[:h - 1, :]], axis=0)
        o_ref[0] = 9.0 * xb - box

    @pl.when(sel == 0)
    def _ident():
        o_ref[0] = xb


def kernel(x):
    b, c, h, w = x.shape
    bc = b * c
    xr = x.reshape(bc, h, w)

    stats = pl.pallas_call(
        _stats_body,
        grid=(bc,),
        in_specs=[pl.BlockSpec((1, h, w), lambda i: (i, 0, 0))],
        out_specs=pl.BlockSpec((1, 1, 16), lambda i: (i, 0, 0)),
        out_shape=jax.ShapeDtypeStruct((bc, 1, 16), jnp.float32),
    )(xr)

    mask = pl.pallas_call(
        _select_body,
        out_shape=jax.ShapeDtypeStruct((b, c), jnp.int32),
    )(stats.reshape(b, c, 16))

    out = pl.pallas_call(
        _conv_body,
        grid=(bc,),
        in_specs=[
            pl.BlockSpec((1, h, w), lambda i: (i, 0, 0)),
            pl.BlockSpec(memory_space=pltpu.SMEM),
        ],
        out_specs=pl.BlockSpec((1, h, w), lambda i: (i, 0, 0)),
        out_shape=jax.ShapeDtypeStruct((bc, h, w), jnp.float32),
    )(xr, mask.reshape(bc))

    return out.reshape(b, c, h, w)


# speedup vs baseline: 2.0576x; 1.8933x over previous
import functools

import jax
import jax.numpy as jnp
from jax.experimental import pallas as pl
from jax.experimental.pallas import tpu as pltpu

_BINS = 10
_K = 10
_G = 8


def _stats_body(x_ref, s_ref):
    xb = x_ref[0]
    g, hh, ww = xb.shape
    n = hh * ww
    s = jnp.sum(xb, axis=(1, 2), keepdims=True)
    mean = s / n
    d = xb - mean
    var = jnp.sum(d * d, axis=(1, 2), keepdims=True) / (n - 1)
    q = jnp.round(jnp.tanh(xb) * 10.0)
    qb = q.astype(jnp.bfloat16)
    lane = jax.lax.broadcasted_iota(jnp.int32, (g, 16), 1)
    vec = jnp.where(lane == 0, var[:, :, 0], 0.0)
    one = jnp.bfloat16(1.0)
    zero = jnp.bfloat16(0.0)
    for v in range(_BINS + 1):
        m = jnp.where(qb == jnp.bfloat16(v), one, zero)
        part = jnp.sum(m, axis=1)
        cnt = jnp.sum(part.astype(jnp.float32), axis=1, keepdims=True)
        vec = jnp.where(lane == 1 + v, cnt, vec)
    s_ref[0] = vec


def _select_body(st_ref, m_ref):
    st = st_ref[...]
    b, c, _ = st.shape
    var = st[:, :, 0]
    hs = st[:, :, 1:_BINS + 2]
    total = jnp.sum(hs, axis=2, keepdims=True)
    p = hs / total
    min_real = jnp.finfo(jnp.float32).min
    logit = jnp.maximum(jnp.log(p), min_real)
    ent = -jnp.sum(logit * p, axis=2)
    score = ent + 2.0 / (var + 1e-7)
    si = score[:, :, None]
    sj = score[:, None, :]
    jidx = jax.lax.broadcasted_iota(jnp.int32, (b, c, c), 2)
    iidx = jax.lax.broadcasted_iota(jnp.int32, (b, c, c), 1)
    ahead = (sj > si) | ((sj == si) & (jidx < iidx))
    rank = jnp.sum(ahead.astype(jnp.int32), axis=2)
    m_ref[...] = (rank < _K).astype(jnp.int32)


def _conv_body(x_ref, m_ref, o_ref):
    pid = pl.program_id(0)
    xb = x_ref[0]
    g, hh, ww = xb.shape
    sub = jax.lax.broadcasted_iota(jnp.int32, (g, 1, 1), 0)
    mv = jnp.zeros((g, 1, 1), jnp.int32)
    for i in range(g):
        mv = jnp.where(sub == i, m_ref[pid * g + i], mv)
    zc = jnp.zeros((g, hh, 1), jnp.float32)
    hsum = xb + jnp.concatenate([xb[:, :, 1:], zc], axis=2) \
              + jnp.concatenate([zc, xb[:, :, :ww - 1]], axis=2)
    zr = jnp.zeros((g, 1, ww), jnp.float32)
    box = hsum + jnp.concatenate([hsum[:, 1:, :], zr], axis=1) \
               + jnp.concatenate([zr, hsum[:, :hh - 1, :]], axis=1)
    lap = 9.0 * xb - box
    o_ref[0] = jnp.where(mv > 0, lap, xb)


def kernel(x):
    b, c, h, w = x.shape
    bc = b * c
    ng = bc // _G
    xg = x.reshape(ng, _G, h, w)

    stats = pl.pallas_call(
        _stats_body,
        grid=(ng,),
        in_specs=[pl.BlockSpec((1, _G, h, w), lambda i: (i, 0, 0, 0))],
        out_specs=pl.BlockSpec((1, _G, 16), lambda i: (i, 0, 0)),
        out_shape=jax.ShapeDtypeStruct((ng, _G, 16), jnp.float32),
    )(xg)

    mask = pl.pallas_call(
        _select_body,
        out_shape=jax.ShapeDtypeStruct((b, c), jnp.int32),
    )(stats.reshape(b, c, 16))

    out = pl.pallas_call(
        _conv_body,
        grid=(ng,),
        in_specs=[
            pl.BlockSpec((1, _G, h, w), lambda i: (i, 0, 0, 0)),
            pl.BlockSpec(memory_space=pltpu.SMEM),
        ],
        out_specs=pl.BlockSpec((1, _G, h, w), lambda i: (i, 0, 0, 0)),
        out_shape=jax.ShapeDtypeStruct((ng, _G, h, w), jnp.float32),
    )(xg, mask.reshape(bc))

    return out.reshape(b, c, h, w)


# speedup vs baseline: 3.3488x; 1.6275x over previous
import functools

import jax
import jax.numpy as jnp
from jax.experimental import pallas as pl
from jax.experimental.pallas import tpu as pltpu

_BINS = 10
_K = 10
_G = 8


def _stats_body(x_ref, s_ref):
    xb = x_ref[0]
    g, hh, ww = xb.shape
    n = hh * ww
    s = jnp.sum(xb, axis=(1, 2), keepdims=True)
    mean = s / n
    d = xb - mean
    var = jnp.sum(d * d, axis=(1, 2), keepdims=True) / (n - 1)
    lane = jax.lax.broadcasted_iota(jnp.int32, (g, 16), 1)
    vec = jnp.where(lane == 0, var[:, :, 0], 0.0)
    rows = 8
    ngrp = hh // rows
    one = jnp.ones((g, rows, ww), jnp.int32)
    zero = jnp.zeros((g, rows, ww), jnp.int32)
    acc1 = zero
    acc2 = zero
    for r in range(ngrp):
        q = jnp.round(jnp.tanh(xb[:, r * rows:(r + 1) * rows, :]) * 10.0)
        qi = q.astype(jnp.int32)
        m1 = (qi >= 0) & (qi < 5)
        m2 = qi >= 5
        q1 = jnp.where(m1, qi, 0)
        q2 = jnp.where(m2, qi - 5, 0)
        acc1 = acc1 + jnp.where(m1, one << (q1 * 6), zero)
        acc2 = acc2 + jnp.where(m2, one << (q2 * 5), zero)
    for v in range(_BINS + 1):
        if v < 5:
            field = (acc1 >> (6 * v)) & 63
        else:
            field = (acc2 >> (5 * (v - 5))) & 31
        cnt = jnp.sum(field.astype(jnp.float32), axis=(1, 2))
        vec = jnp.where(lane == 1 + v, cnt[:, None], vec)
    s_ref[0] = vec


def _select_body(st_ref, m_ref):
    st = st_ref[...]
    b, c, _ = st.shape
    var = st[:, :, 0]
    hs = st[:, :, 1:_BINS + 2]
    total = jnp.sum(hs, axis=2, keepdims=True)
    p = hs / total
    min_real = jnp.finfo(jnp.float32).min
    logit = jnp.maximum(jnp.log(p), min_real)
    ent = -jnp.sum(logit * p, axis=2)
    score = ent + 2.0 / (var + 1e-7)
    si = score[:, :, None]
    sj = score[:, None, :]
    jidx = jax.lax.broadcasted_iota(jnp.int32, (b, c, c), 2)
    iidx = jax.lax.broadcasted_iota(jnp.int32, (b, c, c), 1)
    ahead = (sj > si) | ((sj == si) & (jidx < iidx))
    rank = jnp.sum(ahead.astype(jnp.int32), axis=2)
    m_ref[...] = (rank < _K).astype(jnp.int32)


def _conv_body(x_ref, m_ref, o_ref):
    pid = pl.program_id(0)
    xb = x_ref[0]
    g, hh, ww = xb.shape
    sub = jax.lax.broadcasted_iota(jnp.int32, (g, 1, 1), 0)
    mv = jnp.zeros((g, 1, 1), jnp.int32)
    for i in range(g):
        mv = jnp.where(sub == i, m_ref[pid * g + i], mv)
    zc = jnp.zeros((g, hh, 1), jnp.float32)
    hsum = xb + jnp.concatenate([xb[:, :, 1:], zc], axis=2) \
              + jnp.concatenate([zc, xb[:, :, :ww - 1]], axis=2)
    zr = jnp.zeros((g, 1, ww), jnp.float32)
    box = hsum + jnp.concatenate([hsum[:, 1:, :], zr], axis=1) \
               + jnp.concatenate([zr, hsum[:, :hh - 1, :]], axis=1)
    lap = 9.0 * xb - box
    o_ref[0] = jnp.where(mv > 0, lap, xb)


def kernel(x):
    b, c, h, w = x.shape
    bc = b * c
    ng = bc // _G
    xg = x.reshape(ng, _G, h, w)

    stats = pl.pallas_call(
        _stats_body,
        grid=(ng,),
        in_specs=[pl.BlockSpec((1, _G, h, w), lambda i: (i, 0, 0, 0))],
        out_specs=pl.BlockSpec((1, _G, 16), lambda i: (i, 0, 0)),
        out_shape=jax.ShapeDtypeStruct((ng, _G, 16), jnp.float32),
    )(xg)

    mask = pl.pallas_call(
        _select_body,
        out_shape=jax.ShapeDtypeStruct((b, c), jnp.int32),
    )(stats.reshape(b, c, 16))

    out = pl.pallas_call(
        _conv_body,
        grid=(ng,),
        in_specs=[
            pl.BlockSpec((1, _G, h, w), lambda i: (i, 0, 0, 0)),
            pl.BlockSpec(memory_space=pltpu.SMEM),
        ],
        out_specs=pl.BlockSpec((1, _G, h, w), lambda i: (i, 0, 0, 0)),
        out_shape=jax.ShapeDtypeStruct((ng, _G, h, w), jnp.float32),
    )(xg, mask.reshape(bc))

    return out.reshape(b, c, h, w)


# speedup vs baseline: 3.6594x; 1.0927x over previous
import functools

import jax
import jax.numpy as jnp
from jax.experimental import pallas as pl
from jax.experimental.pallas import tpu as pltpu

_BINS = 10
_K = 10
_G = 16


def _stats_body(x_ref, s_ref):
    xb = x_ref[0]
    g, hh, ww = xb.shape
    n = hh * ww
    s = jnp.sum(xb, axis=(1, 2), keepdims=True)
    mean = s / n
    d = xb - mean
    var = jnp.sum(d * d, axis=(1, 2), keepdims=True) / (n - 1)
    lane = jax.lax.broadcasted_iota(jnp.int32, (g, 16), 1)
    vec = jnp.where(lane == 0, var[:, :, 0], 0.0)
    rows = 8
    ngrp = hh // rows
    one = jnp.ones((g, rows, ww), jnp.int32)
    zero = jnp.zeros((g, rows, ww), jnp.int32)
    acc1 = zero
    acc2 = zero
    for r in range(ngrp):
        q = jnp.round(jnp.tanh(xb[:, r * rows:(r + 1) * rows, :]) * 10.0)
        qi = q.astype(jnp.int32)
        m1 = (qi >= 0) & (qi < 5)
        m2 = qi >= 5
        q1 = jnp.where(m1, qi, 0)
        q2 = jnp.where(m2, qi - 5, 0)
        acc1 = acc1 + jnp.where(m1, one << (q1 * 6), zero)
        acc2 = acc2 + jnp.where(m2, one << (q2 * 5), zero)
    for v in range(_BINS + 1):
        if v < 5:
            field = (acc1 >> (6 * v)) & 63
        else:
            field = (acc2 >> (5 * (v - 5))) & 31
        cnt = jnp.sum(field.astype(jnp.float32), axis=(1, 2))
        vec = jnp.where(lane == 1 + v, cnt[:, None], vec)
    s_ref[0] = vec


def _select_body(st_ref, m_ref):
    st = st_ref[...]
    b, c, _ = st.shape
    var = st[:, :, 0]
    hs = st[:, :, 1:_BINS + 2]
    total = jnp.sum(hs, axis=2, keepdims=True)
    p = hs / total
    min_real = jnp.finfo(jnp.float32).min
    logit = jnp.maximum(jnp.log(p), min_real)
    ent = -jnp.sum(logit * p, axis=2)
    score = ent + 2.0 / (var + 1e-7)
    si = score[:, :, None]
    sj = score[:, None, :]
    jidx = jax.lax.broadcasted_iota(jnp.int32, (b, c, c), 2)
    iidx = jax.lax.broadcasted_iota(jnp.int32, (b, c, c), 1)
    ahead = (sj > si) | ((sj == si) & (jidx < iidx))
    rank = jnp.sum(ahead.astype(jnp.int32), axis=2)
    m_ref[...] = (rank < _K).astype(jnp.int32)


def _conv_body(x_ref, m_ref, o_ref):
    pid = pl.program_id(0)
    xb = x_ref[0]
    g, hh, ww = xb.shape
    sub = jax.lax.broadcasted_iota(jnp.int32, (g, 1, 1), 0)
    mv = jnp.zeros((g, 1, 1), jnp.int32)
    for i in range(g):
        mv = jnp.where(sub == i, m_ref[pid * g + i], mv)
    zc = jnp.zeros((g, hh, 1), jnp.float32)
    hsum = xb + jnp.concatenate([xb[:, :, 1:], zc], axis=2) \
              + jnp.concatenate([zc, xb[:, :, :ww - 1]], axis=2)
    zr = jnp.zeros((g, 1, ww), jnp.float32)
    box = hsum + jnp.concatenate([hsum[:, 1:, :], zr], axis=1) \
               + jnp.concatenate([zr, hsum[:, :hh - 1, :]], axis=1)
    lap = 9.0 * xb - box
    o_ref[0] = jnp.where(mv > 0, lap, xb)


def kernel(x):
    b, c, h, w = x.shape
    bc = b * c
    ng = bc // _G
    xg = x.reshape(ng, _G, h, w)

    stats = pl.pallas_call(
        _stats_body,
        grid=(ng,),
        in_specs=[pl.BlockSpec((1, _G, h, w), lambda i: (i, 0, 0, 0))],
        out_specs=pl.BlockSpec((1, _G, 16), lambda i: (i, 0, 0)),
        out_shape=jax.ShapeDtypeStruct((ng, _G, 16), jnp.float32),
    )(xg)

    mask = pl.pallas_call(
        _select_body,
        out_shape=jax.ShapeDtypeStruct((b, c), jnp.int32),
    )(stats.reshape(b, c, 16))

    out = pl.pallas_call(
        _conv_body,
        grid=(ng,),
        in_specs=[
            pl.BlockSpec((1, _G, h, w), lambda i: (i, 0, 0, 0)),
            pl.BlockSpec(memory_space=pltpu.SMEM),
        ],
        out_specs=pl.BlockSpec((1, _G, h, w), lambda i: (i, 0, 0, 0)),
        out_shape=jax.ShapeDtypeStruct((ng, _G, h, w), jnp.float32),
    )(xg, mask.reshape(bc))

    return out.reshape(b, c, h, w)


# speedup vs baseline: 3.7767x; 1.0321x over previous
import functools

import jax
import jax.numpy as jnp
from jax.experimental import pallas as pl
from jax.experimental.pallas import tpu as pltpu

_BINS = 10
_K = 10
_G = 32


def _stats_body(x_ref, s_ref):
    xb = x_ref[0]
    g, hh, ww = xb.shape
    n = hh * ww
    s = jnp.sum(xb, axis=(1, 2), keepdims=True)
    mean = s / n
    d = xb - mean
    var = jnp.sum(d * d, axis=(1, 2), keepdims=True) / (n - 1)
    lane = jax.lax.broadcasted_iota(jnp.int32, (g, 16), 1)
    vec = jnp.where(lane == 0, var[:, :, 0], 0.0)
    rows = 8
    ngrp = hh // rows
    one = jnp.ones((g, rows, ww), jnp.int32)
    zero = jnp.zeros((g, rows, ww), jnp.int32)
    acc1 = zero
    acc2 = zero
    for r in range(ngrp):
        q = jnp.round(jnp.tanh(xb[:, r * rows:(r + 1) * rows, :]) * 10.0)
        qi = q.astype(jnp.int32)
        m1 = (qi >= 0) & (qi < 5)
        m2 = qi >= 5
        q1 = jnp.where(m1, qi, 0)
        q2 = jnp.where(m2, qi - 5, 0)
        acc1 = acc1 + jnp.where(m1, one << (q1 * 6), zero)
        acc2 = acc2 + jnp.where(m2, one << (q2 * 5), zero)
    for v in range(_BINS + 1):
        if v < 5:
            field = (acc1 >> (6 * v)) & 63
        else:
            field = (acc2 >> (5 * (v - 5))) & 31
        cnt = jnp.sum(field.astype(jnp.float32), axis=(1, 2))
        vec = jnp.where(lane == 1 + v, cnt[:, None], vec)
    s_ref[0] = vec


def _select_body(st_ref, m_ref):
    st = st_ref[...]
    b, c, _ = st.shape
    var = st[:, :, 0]
    hs = st[:, :, 1:_BINS + 2]
    total = jnp.sum(hs, axis=2, keepdims=True)
    p = hs / total
    min_real = jnp.finfo(jnp.float32).min
    logit = jnp.maximum(jnp.log(p), min_real)
    ent = -jnp.sum(logit * p, axis=2)
    score = ent + 2.0 / (var + 1e-7)
    si = score[:, :, None]
    sj = score[:, None, :]
    jidx = jax.lax.broadcasted_iota(jnp.int32, (b, c, c), 2)
    iidx = jax.lax.broadcasted_iota(jnp.int32, (b, c, c), 1)
    ahead = (sj > si) | ((sj == si) & (jidx < iidx))
    rank = jnp.sum(ahead.astype(jnp.int32), axis=2)
    m_ref[...] = (rank < _K).astype(jnp.int32)


def _conv_body(x_ref, m_ref, o_ref):
    pid = pl.program_id(0)
    xb = x_ref[0]
    g, hh, ww = xb.shape
    sub = jax.lax.broadcasted_iota(jnp.int32, (g, 1, 1), 0)
    mv = jnp.zeros((g, 1, 1), jnp.int32)
    for i in range(g):
        mv = jnp.where(sub == i, m_ref[pid * g + i], mv)
    zc = jnp.zeros((g, hh, 1), jnp.float32)
    hsum = xb + jnp.concatenate([xb[:, :, 1:], zc], axis=2) \
              + jnp.concatenate([zc, xb[:, :, :ww - 1]], axis=2)
    zr = jnp.zeros((g, 1, ww), jnp.float32)
    box = hsum + jnp.concatenate([hsum[:, 1:, :], zr], axis=1) \
               + jnp.concatenate([zr, hsum[:, :hh - 1, :]], axis=1)
    lap = 9.0 * xb - box
    o_ref[0] = jnp.where(mv > 0, lap, xb)


def kernel(x):
    b, c, h, w = x.shape
    bc = b * c
    ng = bc // _G
    xg = x.reshape(ng, _G, h, w)

    stats = pl.pallas_call(
        _stats_body,
        grid=(ng,),
        in_specs=[pl.BlockSpec((1, _G, h, w), lambda i: (i, 0, 0, 0))],
        out_specs=pl.BlockSpec((1, _G, 16), lambda i: (i, 0, 0)),
        out_shape=jax.ShapeDtypeStruct((ng, _G, 16), jnp.float32),
    )(xg)

    mask = pl.pallas_call(
        _select_body,
        out_shape=jax.ShapeDtypeStruct((b, c), jnp.int32),
    )(stats.reshape(b, c, 16))

    out = pl.pallas_call(
        _conv_body,
        grid=(ng,),
        in_specs=[
            pl.BlockSpec((1, _G, h, w), lambda i: (i, 0, 0, 0)),
            pl.BlockSpec(memory_space=pltpu.SMEM),
        ],
        out_specs=pl.BlockSpec((1, _G, h, w), lambda i: (i, 0, 0, 0)),
        out_shape=jax.ShapeDtypeStruct((ng, _G, h, w), jnp.float32),
    )(xg, mask.reshape(bc))

    return out.reshape(b, c, h, w)


# speedup vs baseline: 5.1008x; 1.3506x over previous
import functools

import jax
import jax.numpy as jnp
from jax.experimental import pallas as pl
from jax.experimental.pallas import tpu as pltpu

_BINS = 10
_K = 10
_G = 32


def _stats_body(x_ref, s_ref, c_ref):
    xb = x_ref[0]
    g, hh, ww = xb.shape
    n = hh * ww
    c_ref[0] = xb
    s = jnp.sum(xb, axis=(1, 2), keepdims=True)
    mean = s / n
    d = xb - mean
    var = jnp.sum(d * d, axis=(1, 2), keepdims=True) / (n - 1)
    lane = jax.lax.broadcasted_iota(jnp.int32, (g, 16), 1)
    vec = jnp.where(lane == 0, var[:, :, 0], 0.0)
    rows = 8
    ngrp = hh // rows
    one = jnp.ones((g, rows, ww), jnp.int32)
    zero = jnp.zeros((g, rows, ww), jnp.int32)
    acc1 = zero
    acc2 = zero
    for r in range(ngrp):
        q = jnp.round(jnp.tanh(xb[:, r * rows:(r + 1) * rows, :]) * 10.0)
        qi = q.astype(jnp.int32)
        m1 = (qi >= 0) & (qi < 5)
        m2 = qi >= 5
        q1 = jnp.where(m1, qi, 0)
        q2 = jnp.where(m2, qi - 5, 0)
        acc1 = acc1 + jnp.where(m1, one << (q1 * 6), zero)
        acc2 = acc2 + jnp.where(m2, one << (q2 * 5), zero)
    for v in range(_BINS + 1):
        if v < 5:
            field = (acc1 >> (6 * v)) & 63
        else:
            field = (acc2 >> (5 * (v - 5))) & 31
        cnt = jnp.sum(field.astype(jnp.float32), axis=(1, 2))
        vec = jnp.where(lane == 1 + v, cnt[:, None], vec)
    s_ref[0] = vec


def _select_body(st_ref, i_ref):
    st = st_ref[...]
    b, c, _ = st.shape
    var = st[:, :, 0]
    hs = st[:, :, 1:_BINS + 2]
    total = jnp.sum(hs, axis=2, keepdims=True)
    p = hs / total
    min_real = jnp.finfo(jnp.float32).min
    logit = jnp.maximum(jnp.log(p), min_real)
    ent = -jnp.sum(logit * p, axis=2)
    score = ent + 2.0 / (var + 1e-7)
    si = score[:, :, None]
    sj = score[:, None, :]
    jidx = jax.lax.broadcasted_iota(jnp.int32, (b, c, c), 2)
    iidx = jax.lax.broadcasted_iota(jnp.int32, (b, c, c), 1)
    ahead = (sj > si) | ((sj == si) & (jidx < iidx))
    rank = jnp.sum(ahead.astype(jnp.int32), axis=2)
    lane = jax.lax.broadcasted_iota(jnp.int32, (8, 128), 1)
    bj = lane // _K
    rj = lane % _K
    rb = jnp.zeros((128, c), jnp.int32)
    for bb in range(b):
        rb = jnp.where(bj[0, :, None] == bb, rank[bb][None, :], rb)
    on = rb == rj[0, :, None]
    ch = jax.lax.broadcasted_iota(jnp.int32, (128, c), 1)
    idx = jnp.sum(jnp.where(on, ch, 0), axis=1)
    idx = idx + bj[0] * c
    sub = jax.lax.broadcasted_iota(jnp.int32, (8, 128), 0)
    i_ref[...] = jnp.where((sub == 0) & (lane < b * _K), idx[None, :], 0)


def _conv_body(i_ref, x_ref, cp_ref, o_ref):
    del i_ref, cp_ref
    xb = x_ref[0]
    hh, ww = xb.shape
    zc = jnp.zeros((hh, 1), jnp.float32)
    hsum = xb + jnp.concatenate([xb[:, 1:], zc], axis=1) \
              + jnp.concatenate([zc, xb[:, :ww - 1]], axis=1)
    zr = jnp.zeros((1, ww), jnp.float32)
    box = hsum + jnp.concatenate([hsum[1:, :], zr], axis=0) \
               + jnp.concatenate([zr, hsum[:hh - 1, :]], axis=0)
    o_ref[0] = 9.0 * xb - box


def kernel(x):
    b, c, h, w = x.shape
    bc = b * c
    ng = bc // _G
    xg = x.reshape(ng, _G, h, w)

    stats, copy = pl.pallas_call(
        _stats_body,
        grid=(ng,),
        in_specs=[pl.BlockSpec((1, _G, h, w), lambda i: (i, 0, 0, 0))],
        out_specs=[
            pl.BlockSpec((1, _G, 16), lambda i: (i, 0, 0)),
            pl.BlockSpec((1, _G, h, w), lambda i: (i, 0, 0, 0)),
        ],
        out_shape=[
            jax.ShapeDtypeStruct((ng, _G, 16), jnp.float32),
            jax.ShapeDtypeStruct((ng, _G, h, w), jnp.float32),
        ],
    )(xg)

    ind = pl.pallas_call(
        _select_body,
        out_shape=jax.ShapeDtypeStruct((8, 128), jnp.int32),
    )(stats.reshape(b, c, 16))
    ind_flat = ind[0, :b * _K]

    xr = x.reshape(bc, h, w)
    out = pl.pallas_call(
        _conv_body,
        grid_spec=pltpu.PrefetchScalarGridSpec(
            num_scalar_prefetch=1,
            grid=(b * _K,),
            in_specs=[
                pl.BlockSpec((1, h, w), lambda i, ind: (ind[i], 0, 0)),
                pl.BlockSpec(memory_space=pl.ANY),
            ],
            out_specs=pl.BlockSpec((1, h, w), lambda i, ind: (ind[i], 0, 0)),
        ),
        out_shape=jax.ShapeDtypeStruct((bc, h, w), jnp.float32),
        input_output_aliases={2: 0},
    )(ind_flat, xr, copy.reshape(bc, h, w))

    return out.reshape(b, c, h, w)
